# baseline (speedup 1.0000x reference)
import functools

import jax
import jax.numpy as jnp
from jax import lax
from jax.experimental import pallas as pl
from jax.experimental.pallas import tpu as pltpu
from jax.experimental.pallas import tpu_sc as plsc

_D = 64
_S = 2048
_B = 32
_NC = 2
_NS = 16
_NW = _NC * _NS
_PCHUNKS = 8
_PCH = _S // _PCHUNKS
_BGROUPS = _NW // _PCHUNKS
_BPG = _B // _BGROUPS
_NIDX = _PCH // 128


def _sc_embed(idx, token_table, pos_table):
    mesh = plsc.VectorSubcoreMesh(core_axis_name="c", subcore_axis_name="s")

    @functools.partial(
        pl.kernel,
        mesh=mesh,
        compiler_params=pltpu.CompilerParams(use_tc_tiling_on_sc=False),
        out_type=jax.ShapeDtypeStruct((_B, _S, _D), jnp.float32),
        scratch_types=[
            pltpu.VMEM((_NIDX, 128), jnp.int32),
            pltpu.VMEM((_PCH, _D), jnp.float32),
            pltpu.VMEM((_PCH, _D), jnp.float32),
            pltpu.SemaphoreType.DMA,
        ],
    )
    def k(idx_hbm, tok_hbm, pos_hbm, out_hbm, idx_v, rows_v, pos_v, sem):
        wid = lax.axis_index("s") * _NC + lax.axis_index("c")
        p = wid % _PCHUNKS
        g = wid // _PCHUNKS
        pltpu.sync_copy(pos_hbm.at[pl.ds(p * _PCH, _PCH), :], pos_v)

        def step(bl, carry):
            b = g * _BPG + bl
            pltpu.sync_copy(idx_hbm.at[b, pl.ds(p * _NIDX, _NIDX), :], idx_v)
            cps = [
                pltpu.async_copy(
                    tok_hbm.at[idx_v.at[j]],
                    rows_v.at[pl.ds(j * 128, 128)],
                    sem,
                )
                for j in range(_NIDX)
            ]
            for cp in cps:
                cp.wait()

            def add_row(r, c2):
                for j in range(_D // 16):
                    sl = pl.ds(j * 16, 16)
                    rows_v[r, sl] = rows_v[r, sl] + pos_v[r, sl]
                return c2

            lax.fori_loop(0, _PCH, add_row, 0)
            pltpu.sync_copy(rows_v, out_hbm.at[b, pl.ds(p * _PCH, _PCH), :])
            return carry

        lax.fori_loop(0, _BPG, step, 0)

    return k(idx, token_table, pos_table)


def kernel(inputs, token_table, pos_table):
    idx = inputs.astype(jnp.int32).reshape(_B, _S // 128, 128)
    return _sc_embed(idx, token_table, pos_table)
